# baseline (speedup 1.0000x reference)
import functools
import jax
import jax.numpy as jnp
from jax import lax
from jax.experimental import pallas as pl
from jax.experimental.pallas import tpu as pltpu
from jax.experimental.pallas import tpu_sc as plsc

BATCH = 16
HWA = 10647
NC = 80
GRIDS = (52, 26, 13)
OFFS = (0, 2704, 3380)
ANC = ((0.024, 0.031), (0.038, 0.072), (0.079, 0.055), (0.072, 0.147),
       (0.149, 0.108), (0.142, 0.286), (0.279, 0.216), (0.375, 0.476),
       (0.897, 0.784))
LN2 = 0.6931471805599453


def _splat_f(v):
    return jnp.full((16,), v, jnp.float32)


def _splat_i(v):
    return jnp.full((16,), v, jnp.int32)


def _sigmoid(x):
    return 1.0 / (1.0 + jnp.exp(-x))


def _atanh2(t):
    t2 = t * t
    return t * (2.0 + t2 * (2.0 / 3.0 + t2 * (2.0 / 5.0 + t2 * (
        2.0 / 7.0 + t2 * (2.0 / 9.0 + t2 * (2.0 / 11.0))))))


def _log(x):
    bits = plsc.bitcast(x, jnp.int32)
    e = lax.shift_right_logical(bits, _splat_i(23))
    e = jnp.bitwise_and(e, _splat_i(0xFF)) - 127
    mbits = jnp.bitwise_or(jnp.bitwise_and(bits, _splat_i(0x7FFFFF)),
                           _splat_i(0x3F800000))
    m = plsc.bitcast(mbits, jnp.float32)
    big = m > 1.4142135623730951
    m = jnp.where(big, m * 0.5, m)
    e = e + jnp.where(big, _splat_i(1), _splat_i(0))
    t = (m - 1.0) / (m + 1.0)
    return e.astype(jnp.float32) * LN2 + _atanh2(t)


def _softplus_neg_abs(x):
    z = jnp.exp(-jnp.abs(x))
    return _atanh2(z / (z + 2.0))


def _sc_body(pconf_hbm, pcls_hbm, ptx_hbm, gb_hbm, glab_hbm, out_hbm,
             pconf_v, ptx_v, evtbl, gb_v, glab_v, idx_v, pclsr_v, part_v,
             sem):
    lane = lax.iota(jnp.int32, 16)
    lane8 = jnp.minimum(lane, 7)
    valid8 = lane < 8
    wid = lax.axis_index("s") * 2 + lax.axis_index("c")
    part_v[...] = jnp.zeros((16,), jnp.float32)

    @pl.when(wid < BATCH)
    def _():
        b = wid
        pltpu.sync_copy(pconf_hbm.at[b], pconf_v.at[pl.ds(0, HWA)])
        pltpu.sync_copy(ptx_hbm.at[b], ptx_v)
        pltpu.sync_copy(gb_hbm.at[b], gb_v)
        pltpu.sync_copy(glab_hbm.at[b], glab_v)

        bl = plsc.load_gather(gb_v, [lane8 * 4 + 0])
        bt = plsc.load_gather(gb_v, [lane8 * 4 + 1])
        br = plsc.load_gather(gb_v, [lane8 * 4 + 2])
        bb = plsc.load_gather(gb_v, [lane8 * 4 + 3])
        cx = (bl + br) * 0.5
        cy = (bt + bb) * 0.5
        w = br - bl
        h = bb - bt

        masks = []
        best = None
        bestj = _splat_i(0)
        for j in range(9):
            aw, ah = ANC[j]
            inter = jnp.minimum(w, aw) * jnp.minimum(h, ah)
            union = w * h + (aw * ah) - inter + 1e-16
            iou = inter / union
            masks.append(iou > 0.5)
            if best is None:
                best = iou
            else:
                upd = iou > best
                best = jnp.where(upd, iou, best)
                bestj = jnp.where(upd, _splat_i(j), bestj)
        acts = [masks[3 * l] | masks[3 * l + 1] | masks[3 * l + 2]
                for l in range(3)]

        Rs = []
        for l in range(3):
            g = GRIDS[l]
            col = (cx * jnp.float32(g)).astype(jnp.int32)
            row = (cy * jnp.float32(g)).astype(jnp.int32)
            Rs.append((_splat_i(OFFS[l]) + row * g + col) * 3)
        lb = bestj // 3
        gf = jnp.where(lb == 0, _splat_f(52.0),
                       jnp.where(lb == 1, _splat_f(26.0), _splat_f(13.0)))
        gi = jnp.where(lb == 0, _splat_i(52),
                       jnp.where(lb == 1, _splat_i(26), _splat_i(13)))
        offb = jnp.where(lb == 0, _splat_i(OFFS[0]),
                         jnp.where(lb == 1, _splat_i(OFFS[1]),
                                   _splat_i(OFFS[2])))
        colb = (cx * gf).astype(jnp.int32)
        rowb = (cy * gf).astype(jnp.int32)
        Rb = (offb + rowb * gi + colb) * 3

        q0 = (b * HWA + Rb) * NC
        r0 = lax.shift_right_logical(q0, _splat_i(7))
        off0 = jnp.bitwise_and(q0, _splat_i(127))
        plsc.store_scatter(idx_v, [lane * 2], r0)
        plsc.store_scatter(idx_v, [lane * 2 + 1], r0 + 1)
        cp = pltpu.async_copy(pcls_hbm.at[idx_v], pclsr_v, sem)

        evs = [(Rs[0], acts[0]), (Rs[1], acts[1]), (Rs[2], acts[2]),
               (Rb, valid8)]
        for i in range(8):
            onlane = lane == i
            for k in range(4):
                Rk, actk = evs[k]
                plsc.store_scatter(evtbl, [Rk], _splat_i(4 * i + k),
                                   mask=onlane & actk)
        lives = []
        svals = []
        for k in range(4):
            Rk, actk = evs[k]
            winner = plsc.load_gather(evtbl, [Rk])
            lives.append(actk & (winner == lane8 * 4 + k) & valid8)
            svals.append(_sigmoid(plsc.load_gather(pconf_v, [Rk])))

        def dbody(i, acc):
            x = pconf_v[pl.ds(i * 16, 16)]
            s = _sigmoid(x)
            gidx = i * 16 + lane
            return acc + jnp.where(gidx < HWA, s * s, 0.0)

        a_vec = lax.fori_loop(0, (HWA + 15) // 16, dbody,
                              jnp.zeros((16,), jnp.float32))

        for k in range(4):
            a_vec = a_vec - jnp.where(lives[k], svals[k] * svals[k], 0.0)

        live3 = lives[3]
        s3 = svals[3]
        anc_w = _splat_f(ANC[0][0])
        anc_h = _splat_f(ANC[0][1])
        for j in range(1, 9):
            selj = bestj == j
            anc_w = jnp.where(selj, _splat_f(ANC[j][0]), anc_w)
            anc_h = jnp.where(selj, _splat_f(ANC[j][1]), anc_h)
        tx = plsc.load_gather(ptx_v, [Rb * 4 + 0])
        ty = plsc.load_gather(ptx_v, [Rb * 4 + 1])
        tw = plsc.load_gather(ptx_v, [Rb * 4 + 2])
        th = plsc.load_gather(ptx_v, [Rb * 4 + 3])
        colbf = colb.astype(jnp.float32)
        rowbf = rowb.astype(jnp.float32)
        px = (_sigmoid(tx) + colbf) / gf
        py = (_sigmoid(ty) + rowbf) / gf
        pw = jnp.exp(tw) * anc_w
        ph = jnp.exp(th) * anc_h
        pl_x = px - pw * 0.5
        pr_x = px + pw * 0.5
        pl_y = py - ph * 0.5
        pr_y = py + ph * 0.5
        ix = jnp.maximum(jnp.minimum(pr_x, br) - jnp.maximum(pl_x, bl), 0.0)
        iy = jnp.maximum(jnp.minimum(pr_y, bb) - jnp.maximum(pl_y, bt), 0.0)
        inter1 = ix * iy
        area_p = (pr_x - pl_x) * (pr_y - pl_y)
        area_g = (br - bl) * (bb - bt)
        conf = inter1 / (area_p + area_g - inter1 + 1e-16)
        weight = 2.0 - br * bb
        txy_x = (cx - colbf / gf) * gf
        txy_y = (cy - rowbf / gf) * gf
        twh_x = _log(br / anc_w)
        twh_y = _log(bb / anc_h)
        bce_t = (jnp.maximum(tx, 0.0) - tx * txy_x + _softplus_neg_abs(tx)
                 + jnp.maximum(ty, 0.0) - ty * txy_y + _softplus_neg_abs(ty))
        dw = tw - twh_x
        dh = th - twh_y
        sq_t = dw * dw + dh * dh
        d3 = s3 - conf
        pos_corr = 5.0 * d3 * d3 + weight * (bce_t + sq_t)
        a_vec = a_vec + jnp.where(live3, pos_corr, 0.0)
        n_vec = jnp.where(live3, _splat_f(1.0), 0.0)

        cp.wait()
        labm1 = plsc.load_gather(glab_v, [lane8]) - 1
        base = jnp.zeros((16,), jnp.float32)
        for c in range(NC):
            off = off0 + c
            slot = lane8 * 2 + lax.shift_right_logical(off, _splat_i(7))
            col = jnp.bitwise_and(off, _splat_i(127))
            xc = plsc.load_gather(pclsr_v, [slot, col])
            base = base + jnp.maximum(xc, 0.0) + _softplus_neg_abs(xc)
        offl = off0 + labm1
        xlab = plsc.load_gather(
            pclsr_v, [lane8 * 2 + lax.shift_right_logical(offl, _splat_i(7)),
                      jnp.bitwise_and(offl, _splat_i(127))])
        c_vec = jnp.where(live3, base - xlab, 0.0)

        a_s = jnp.sum(a_vec)
        c_s = jnp.sum(c_vec)
        n_s = jnp.sum(n_vec)
        part_v[...] = (jnp.where(lane == 0, a_s, 0.0)
                       + jnp.where(lane == 1, c_s, 0.0)
                       + jnp.where(lane == 2, n_s, 0.0))

    pltpu.sync_copy(part_v, out_hbm.at[wid])


@jax.jit
def kernel(pconf, pcls, ptxywh, gboxes, glabels):
    pconf2 = pconf.reshape(BATCH, HWA)
    pcls2 = jnp.pad(pcls.reshape(BATCH * HWA * NC // 128, 128),
                    ((0, 2), (0, 0)))
    ptx2 = ptxywh.reshape(BATCH, HWA * 4)
    gb2 = gboxes.reshape(BATCH, 32)
    glab = glabels.astype(jnp.int32)

    mesh = plsc.VectorSubcoreMesh(core_axis_name="c", subcore_axis_name="s")
    kfn = functools.partial(
        pl.kernel,
        out_type=jax.ShapeDtypeStruct((32, 16), jnp.float32),
        mesh=mesh,
        compiler_params=pltpu.CompilerParams(needs_layout_passes=False,
                                             use_tc_tiling_on_sc=False),
        scratch_types=[
            pltpu.VMEM((10656,), jnp.float32),
            pltpu.VMEM((HWA * 4,), jnp.float32),
            pltpu.VMEM((10656,), jnp.int32),
            pltpu.VMEM((32,), jnp.float32),
            pltpu.VMEM((8,), jnp.int32),
            pltpu.VMEM((32,), jnp.int32),
            pltpu.VMEM((32, 128), jnp.float32),
            pltpu.VMEM((16,), jnp.float32),
            pltpu.SemaphoreType.DMA,
        ],
    )(_sc_body)
    parts = kfn(pconf2, pcls2, ptx2, gb2, glab)
    p = parts.sum(0)
    return p[0] / BATCH + p[1] / jnp.maximum(p[2], 1.0)


# speedup vs baseline: 1.4546x; 1.4546x over previous
import functools
import jax
import jax.numpy as jnp
from jax import lax
from jax.experimental import pallas as pl
from jax.experimental.pallas import tpu as pltpu
from jax.experimental.pallas import tpu_sc as plsc

BATCH = 16
HWA = 10647
NC = 80
GRIDS = (52, 26, 13)
OFFS = (0, 2704, 3380)
ANC = ((0.024, 0.031), (0.038, 0.072), (0.079, 0.055), (0.072, 0.147),
       (0.149, 0.108), (0.142, 0.286), (0.279, 0.216), (0.375, 0.476),
       (0.897, 0.784))
LN2 = 0.6931471805599453


def _splat_f(v):
    return jnp.full((16,), v, jnp.float32)


def _splat_i(v):
    return jnp.full((16,), v, jnp.int32)


def _sigmoid(x):
    return 1.0 / (1.0 + jnp.exp(-x))


def _atanh2(t):
    t2 = t * t
    return t * (2.0 + t2 * (2.0 / 3.0 + t2 * (2.0 / 5.0 + t2 * (
        2.0 / 7.0 + t2 * (2.0 / 9.0 + t2 * (2.0 / 11.0))))))


def _log(x):
    bits = plsc.bitcast(x, jnp.int32)
    e = lax.shift_right_logical(bits, _splat_i(23))
    e = jnp.bitwise_and(e, _splat_i(0xFF)) - 127
    mbits = jnp.bitwise_or(jnp.bitwise_and(bits, _splat_i(0x7FFFFF)),
                           _splat_i(0x3F800000))
    m = plsc.bitcast(mbits, jnp.float32)
    big = m > 1.4142135623730951
    m = jnp.where(big, m * 0.5, m)
    e = e + jnp.where(big, _splat_i(1), _splat_i(0))
    t = (m - 1.0) / (m + 1.0)
    return e.astype(jnp.float32) * LN2 + _atanh2(t)


def _softplus_neg_abs(x):
    z = jnp.exp(-jnp.abs(x))
    return _atanh2(z / (z + 2.0))


def _sc_body(pconf_hbm, pcls_hbm, ptx_hbm, gb_hbm, glab_hbm, out_hbm,
             pconf_v, evtbl, gb_v, glab_v, idxc_v, idxt_v, pclsr_v, ptr_v,
             part_v, semc, semt):
    lane = lax.iota(jnp.int32, 16)
    lane8 = jnp.minimum(lane, 7)
    valid8 = lane < 8
    wid = lax.axis_index("s") * 2 + lax.axis_index("c")
    part_v[...] = jnp.zeros((16,), jnp.float32)

    @pl.when(wid < BATCH)
    def _():
        b = wid
        pltpu.sync_copy(pconf_hbm.at[b], pconf_v.at[pl.ds(0, HWA)])
        pltpu.sync_copy(gb_hbm.at[b], gb_v)
        pltpu.sync_copy(glab_hbm.at[b], glab_v)

        bl = plsc.load_gather(gb_v, [lane8 * 4 + 0])
        bt = plsc.load_gather(gb_v, [lane8 * 4 + 1])
        br = plsc.load_gather(gb_v, [lane8 * 4 + 2])
        bb = plsc.load_gather(gb_v, [lane8 * 4 + 3])
        cx = (bl + br) * 0.5
        cy = (bt + bb) * 0.5
        w = br - bl
        h = bb - bt

        masks = []
        best = None
        bestj = _splat_i(0)
        for j in range(9):
            aw, ah = ANC[j]
            inter = jnp.minimum(w, aw) * jnp.minimum(h, ah)
            union = w * h + (aw * ah) - inter + 1e-16
            iou = inter / union
            masks.append(iou > 0.5)
            if best is None:
                best = iou
            else:
                upd = iou > best
                best = jnp.where(upd, iou, best)
                bestj = jnp.where(upd, _splat_i(j), bestj)
        acts = [masks[3 * l] | masks[3 * l + 1] | masks[3 * l + 2]
                for l in range(3)]

        Rs = []
        for l in range(3):
            g = GRIDS[l]
            col = (cx * jnp.float32(g)).astype(jnp.int32)
            row = (cy * jnp.float32(g)).astype(jnp.int32)
            Rs.append((_splat_i(OFFS[l]) + row * g + col) * 3)
        lb = bestj // 3
        gf = jnp.where(lb == 0, _splat_f(52.0),
                       jnp.where(lb == 1, _splat_f(26.0), _splat_f(13.0)))
        gi = jnp.where(lb == 0, _splat_i(52),
                       jnp.where(lb == 1, _splat_i(26), _splat_i(13)))
        offb = jnp.where(lb == 0, _splat_i(OFFS[0]),
                         jnp.where(lb == 1, _splat_i(OFFS[1]),
                                   _splat_i(OFFS[2])))
        colb = (cx * gf).astype(jnp.int32)
        rowb = (cy * gf).astype(jnp.int32)
        Rb = (offb + rowb * gi + colb) * 3

        idxc_v[...] = b * HWA + Rb
        cpc = pltpu.async_copy(pcls_hbm.at[idxc_v], pclsr_v, semc)
        qt = (b * HWA + Rb) * 4
        tr0 = lax.shift_right_logical(qt, _splat_i(7))
        toff = jnp.bitwise_and(qt, _splat_i(127))
        plsc.store_scatter(idxt_v, [lane * 2], tr0)
        plsc.store_scatter(idxt_v, [lane * 2 + 1], tr0 + 1)
        cpt = pltpu.async_copy(ptx_hbm.at[idxt_v], ptr_v, semt)

        evs = [(Rs[0], acts[0]), (Rs[1], acts[1]), (Rs[2], acts[2]),
               (Rb, valid8)]
        for i in range(8):
            onlane = lane == i
            for k in range(4):
                Rk, actk = evs[k]
                plsc.store_scatter(evtbl, [Rk], _splat_i(4 * i + k),
                                   mask=onlane & actk)
        lives = []
        svals = []
        for k in range(4):
            Rk, actk = evs[k]
            winner = plsc.load_gather(evtbl, [Rk])
            lives.append(actk & (winner == lane8 * 4 + k) & valid8)
            svals.append(_sigmoid(plsc.load_gather(pconf_v, [Rk])))

        def dbody(i, acc):
            x = pconf_v[pl.ds(i * 16, 16)]
            s = _sigmoid(x)
            gidx = i * 16 + lane
            return acc + jnp.where(gidx < HWA, s * s, 0.0)

        a_vec = lax.fori_loop(0, (HWA + 15) // 16, dbody,
                              jnp.zeros((16,), jnp.float32))

        for k in range(4):
            a_vec = a_vec - jnp.where(lives[k], svals[k] * svals[k], 0.0)

        live3 = lives[3]
        s3 = svals[3]
        anc_w = _splat_f(ANC[0][0])
        anc_h = _splat_f(ANC[0][1])
        for j in range(1, 9):
            selj = bestj == j
            anc_w = jnp.where(selj, _splat_f(ANC[j][0]), anc_w)
            anc_h = jnp.where(selj, _splat_f(ANC[j][1]), anc_h)
        cpt.wait()

        def ptx_at(c):
            off = toff + c
            return plsc.load_gather(
                ptr_v, [lane8 * 2 + lax.shift_right_logical(off, _splat_i(7)),
                        jnp.bitwise_and(off, _splat_i(127))])

        tx = ptx_at(0)
        ty = ptx_at(1)
        tw = ptx_at(2)
        th = ptx_at(3)
        colbf = colb.astype(jnp.float32)
        rowbf = rowb.astype(jnp.float32)
        px = (_sigmoid(tx) + colbf) / gf
        py = (_sigmoid(ty) + rowbf) / gf
        pw = jnp.exp(tw) * anc_w
        ph = jnp.exp(th) * anc_h
        pl_x = px - pw * 0.5
        pr_x = px + pw * 0.5
        pl_y = py - ph * 0.5
        pr_y = py + ph * 0.5
        ix = jnp.maximum(jnp.minimum(pr_x, br) - jnp.maximum(pl_x, bl), 0.0)
        iy = jnp.maximum(jnp.minimum(pr_y, bb) - jnp.maximum(pl_y, bt), 0.0)
        inter1 = ix * iy
        area_p = (pr_x - pl_x) * (pr_y - pl_y)
        area_g = (br - bl) * (bb - bt)
        conf = inter1 / (area_p + area_g - inter1 + 1e-16)
        weight = 2.0 - br * bb
        txy_x = (cx - colbf / gf) * gf
        txy_y = (cy - rowbf / gf) * gf
        twh_x = _log(br / anc_w)
        twh_y = _log(bb / anc_h)
        bce_t = (jnp.maximum(tx, 0.0) - tx * txy_x + _softplus_neg_abs(tx)
                 + jnp.maximum(ty, 0.0) - ty * txy_y + _softplus_neg_abs(ty))
        dw = tw - twh_x
        dh = th - twh_y
        sq_t = dw * dw + dh * dh
        d3 = s3 - conf
        pos_corr = 5.0 * d3 * d3 + weight * (bce_t + sq_t)
        a_vec = a_vec + jnp.where(live3, pos_corr, 0.0)
        n_vec = jnp.where(live3, _splat_f(1.0), 0.0)

        cpc.wait()
        labm1 = plsc.load_gather(glab_v, [lane8]) - 1
        base = jnp.zeros((16,), jnp.float32)
        for c in range(NC):
            xc = plsc.load_gather(pclsr_v, [lane, _splat_i(c)])
            base = base + jnp.maximum(xc, 0.0) + _softplus_neg_abs(xc)
        xlab = plsc.load_gather(pclsr_v, [lane, labm1])
        c_vec = jnp.where(live3, base - xlab, 0.0)

        a_s = jnp.sum(a_vec)
        c_s = jnp.sum(c_vec)
        n_s = jnp.sum(n_vec)
        part_v[...] = (jnp.where(lane == 0, a_s, 0.0)
                       + jnp.where(lane == 1, c_s, 0.0)
                       + jnp.where(lane == 2, n_s, 0.0))

    pltpu.sync_copy(part_v, out_hbm.at[wid])


@jax.jit
def kernel(pconf, pcls, ptxywh, gboxes, glabels):
    pconf2 = pconf.reshape(BATCH, HWA)
    pcls2 = pcls.reshape(BATCH * HWA, NC)
    ptx2 = jnp.pad(
        jnp.pad(ptxywh.reshape(-1), (0, 64)).reshape(5324, 128),
        ((0, 4), (0, 0)))
    gb2 = gboxes.reshape(BATCH, 32)
    glab = glabels.astype(jnp.int32)

    mesh = plsc.VectorSubcoreMesh(core_axis_name="c", subcore_axis_name="s")
    kfn = functools.partial(
        pl.kernel,
        out_type=jax.ShapeDtypeStruct((32, 16), jnp.float32),
        mesh=mesh,
        compiler_params=pltpu.CompilerParams(needs_layout_passes=False,
                                             use_tc_tiling_on_sc=False),
        scratch_types=[
            pltpu.VMEM((10656,), jnp.float32),
            pltpu.VMEM((10656,), jnp.int32),
            pltpu.VMEM((32,), jnp.float32),
            pltpu.VMEM((8,), jnp.int32),
            pltpu.VMEM((16,), jnp.int32),
            pltpu.VMEM((32,), jnp.int32),
            pltpu.VMEM((16, NC), jnp.float32),
            pltpu.VMEM((32, 128), jnp.float32),
            pltpu.VMEM((16,), jnp.float32),
            pltpu.SemaphoreType.DMA,
            pltpu.SemaphoreType.DMA,
        ],
    )(_sc_body)
    parts = kfn(pconf2, pcls2, ptx2, gb2, glab)
    p = parts.sum(0)
    return p[0] / BATCH + p[1] / jnp.maximum(p[2], 1.0)


# speedup vs baseline: 1.7239x; 1.1851x over previous
import functools
import jax
import jax.numpy as jnp
from jax import lax
from jax.experimental import pallas as pl
from jax.experimental.pallas import tpu as pltpu
from jax.experimental.pallas import tpu_sc as plsc

BATCH = 16
HWA = 10647
NC = 80
GRIDS = (52, 26, 13)
OFFS = (0, 2704, 3380)
ANC = ((0.024, 0.031), (0.038, 0.072), (0.079, 0.055), (0.072, 0.147),
       (0.149, 0.108), (0.142, 0.286), (0.279, 0.216), (0.375, 0.476),
       (0.897, 0.784))
LN2 = 0.6931471805599453


def _splat_f(v):
    return jnp.full((16,), v, jnp.float32)


def _splat_i(v):
    return jnp.full((16,), v, jnp.int32)


def _sigmoid(x):
    return 1.0 / (1.0 + jnp.exp(-x))


def _atanh2(t):
    t2 = t * t
    return t * (2.0 + t2 * (2.0 / 3.0 + t2 * (2.0 / 5.0 + t2 * (
        2.0 / 7.0 + t2 * (2.0 / 9.0 + t2 * (2.0 / 11.0))))))


def _log(x):
    bits = plsc.bitcast(x, jnp.int32)
    e = lax.shift_right_logical(bits, _splat_i(23))
    e = jnp.bitwise_and(e, _splat_i(0xFF)) - 127
    mbits = jnp.bitwise_or(jnp.bitwise_and(bits, _splat_i(0x7FFFFF)),
                           _splat_i(0x3F800000))
    m = plsc.bitcast(mbits, jnp.float32)
    big = m > 1.4142135623730951
    m = jnp.where(big, m * 0.5, m)
    e = e + jnp.where(big, _splat_i(1), _splat_i(0))
    t = (m - 1.0) / (m + 1.0)
    return e.astype(jnp.float32) * LN2 + _atanh2(t)


def _softplus_neg_abs(x):
    z = jnp.exp(-jnp.abs(x))
    return _atanh2(z / (z + 2.0))


def _sc_body(pconf_hbm, pcls_hbm, ptx_hbm, gb_hbm, glab_hbm, out_hbm,
             pconf_v, evtbl, gb_v, glab_v, idxc_v, idxt_v, pclsr_v, ptr_v,
             part_v, semc, semt):
    lane = lax.iota(jnp.int32, 16)
    lane8 = jnp.minimum(lane, 7)
    valid8 = lane < 8
    wid = lax.axis_index("s") * 2 + lax.axis_index("c")
    part_v[...] = jnp.zeros((16,), jnp.float32)

    @pl.when(wid < BATCH)
    def _():
        b = wid
        pltpu.sync_copy(pconf_hbm.at[b], pconf_v.at[pl.ds(0, HWA)])
        pltpu.sync_copy(gb_hbm.at[b], gb_v)
        pltpu.sync_copy(glab_hbm.at[b], glab_v)

        bl = plsc.load_gather(gb_v, [lane8 * 4 + 0])
        bt = plsc.load_gather(gb_v, [lane8 * 4 + 1])
        br = plsc.load_gather(gb_v, [lane8 * 4 + 2])
        bb = plsc.load_gather(gb_v, [lane8 * 4 + 3])
        cx = (bl + br) * 0.5
        cy = (bt + bb) * 0.5
        w = br - bl
        h = bb - bt

        masks = []
        best = None
        bestj = _splat_i(0)
        for j in range(9):
            aw, ah = ANC[j]
            inter = jnp.minimum(w, aw) * jnp.minimum(h, ah)
            union = w * h + (aw * ah) - inter + 1e-16
            iou = inter / union
            masks.append(iou > 0.5)
            if best is None:
                best = iou
            else:
                upd = iou > best
                best = jnp.where(upd, iou, best)
                bestj = jnp.where(upd, _splat_i(j), bestj)
        acts = [masks[3 * l] | masks[3 * l + 1] | masks[3 * l + 2]
                for l in range(3)]

        Rs = []
        for l in range(3):
            g = GRIDS[l]
            col = (cx * jnp.float32(g)).astype(jnp.int32)
            row = (cy * jnp.float32(g)).astype(jnp.int32)
            Rs.append((_splat_i(OFFS[l]) + row * g + col) * 3)
        lb = bestj // 3
        gf = jnp.where(lb == 0, _splat_f(52.0),
                       jnp.where(lb == 1, _splat_f(26.0), _splat_f(13.0)))
        gi = jnp.where(lb == 0, _splat_i(52),
                       jnp.where(lb == 1, _splat_i(26), _splat_i(13)))
        offb = jnp.where(lb == 0, _splat_i(OFFS[0]),
                         jnp.where(lb == 1, _splat_i(OFFS[1]),
                                   _splat_i(OFFS[2])))
        colb = (cx * gf).astype(jnp.int32)
        rowb = (cy * gf).astype(jnp.int32)
        Rb = (offb + rowb * gi + colb) * 3

        wbase = b * (NC * HWA) + Rb
        for c in range(NC):
            plsc.store_scatter(
                idxc_v, [lane + c * 16],
                lax.shift_right_logical(wbase + c * HWA, _splat_i(3)))
        cpc = pltpu.async_copy(pcls_hbm.at[idxc_v], pclsr_v, semc)
        qt = (b * HWA + Rb) * 4
        tr0 = lax.shift_right_logical(qt, _splat_i(7))
        toff = jnp.bitwise_and(qt, _splat_i(127))
        plsc.store_scatter(idxt_v, [lane * 2], tr0)
        plsc.store_scatter(idxt_v, [lane * 2 + 1], tr0 + 1)
        cpt = pltpu.async_copy(ptx_hbm.at[idxt_v], ptr_v, semt)

        evs = [(Rs[0], acts[0]), (Rs[1], acts[1]), (Rs[2], acts[2]),
               (Rb, valid8)]
        for i in range(8):
            onlane = lane == i
            for k in range(4):
                Rk, actk = evs[k]
                plsc.store_scatter(evtbl, [Rk], _splat_i(4 * i + k),
                                   mask=onlane & actk)
        lives = []
        svals = []
        for k in range(4):
            Rk, actk = evs[k]
            winner = plsc.load_gather(evtbl, [Rk])
            lives.append(actk & (winner == lane8 * 4 + k) & valid8)
            svals.append(_sigmoid(plsc.load_gather(pconf_v, [Rk])))

        def dbody(i, acc):
            x = pconf_v[pl.ds(i * 16, 16)]
            s = _sigmoid(x)
            gidx = i * 16 + lane
            return acc + jnp.where(gidx < HWA, s * s, 0.0)

        a_vec = lax.fori_loop(0, (HWA + 15) // 16, dbody,
                              jnp.zeros((16,), jnp.float32))

        for k in range(4):
            a_vec = a_vec - jnp.where(lives[k], svals[k] * svals[k], 0.0)

        live3 = lives[3]
        s3 = svals[3]
        anc_w = _splat_f(ANC[0][0])
        anc_h = _splat_f(ANC[0][1])
        for j in range(1, 9):
            selj = bestj == j
            anc_w = jnp.where(selj, _splat_f(ANC[j][0]), anc_w)
            anc_h = jnp.where(selj, _splat_f(ANC[j][1]), anc_h)
        cpt.wait()

        def ptx_at(c):
            off = toff + c
            return plsc.load_gather(
                ptr_v, [lane8 * 2 + lax.shift_right_logical(off, _splat_i(7)),
                        jnp.bitwise_and(off, _splat_i(127))])

        tx = ptx_at(0)
        ty = ptx_at(1)
        tw = ptx_at(2)
        th = ptx_at(3)
        colbf = colb.astype(jnp.float32)
        rowbf = rowb.astype(jnp.float32)
        px = (_sigmoid(tx) + colbf) / gf
        py = (_sigmoid(ty) + rowbf) / gf
        pw = jnp.exp(tw) * anc_w
        ph = jnp.exp(th) * anc_h
        pl_x = px - pw * 0.5
        pr_x = px + pw * 0.5
        pl_y = py - ph * 0.5
        pr_y = py + ph * 0.5
        ix = jnp.maximum(jnp.minimum(pr_x, br) - jnp.maximum(pl_x, bl), 0.0)
        iy = jnp.maximum(jnp.minimum(pr_y, bb) - jnp.maximum(pl_y, bt), 0.0)
        inter1 = ix * iy
        area_p = (pr_x - pl_x) * (pr_y - pl_y)
        area_g = (br - bl) * (bb - bt)
        conf = inter1 / (area_p + area_g - inter1 + 1e-16)
        weight = 2.0 - br * bb
        txy_x = (cx - colbf / gf) * gf
        txy_y = (cy - rowbf / gf) * gf
        twh_x = _log(br / anc_w)
        twh_y = _log(bb / anc_h)
        bce_t = (jnp.maximum(tx, 0.0) - tx * txy_x + _softplus_neg_abs(tx)
                 + jnp.maximum(ty, 0.0) - ty * txy_y + _softplus_neg_abs(ty))
        dw = tw - twh_x
        dh = th - twh_y
        sq_t = dw * dw + dh * dh
        d3 = s3 - conf
        pos_corr = 5.0 * d3 * d3 + weight * (bce_t + sq_t)
        a_vec = a_vec + jnp.where(live3, pos_corr, 0.0)
        n_vec = jnp.where(live3, _splat_f(1.0), 0.0)

        cpc.wait()
        labm1 = plsc.load_gather(glab_v, [lane8]) - 1
        base = jnp.zeros((16,), jnp.float32)
        for c in range(NC):
            wc = wbase + c * HWA
            xc = plsc.load_gather(
                pclsr_v, [lane + c * 16, jnp.bitwise_and(wc, _splat_i(7))])
            base = base + jnp.maximum(xc, 0.0) + _softplus_neg_abs(xc)
        wl = wbase + labm1 * HWA
        xlab = plsc.load_gather(
            pclsr_v, [lane + labm1 * 16, jnp.bitwise_and(wl, _splat_i(7))])
        c_vec = jnp.where(live3, base - xlab, 0.0)

        a_s = jnp.sum(a_vec)
        c_s = jnp.sum(c_vec)
        n_s = jnp.sum(n_vec)
        part_v[...] = (jnp.where(lane == 0, a_s, 0.0)
                       + jnp.where(lane == 1, c_s, 0.0)
                       + jnp.where(lane == 2, n_s, 0.0))

    pltpu.sync_copy(part_v, out_hbm.at[wid])


@jax.jit
def kernel(pconf, pcls, ptxywh, gboxes, glabels):
    pconf2 = pconf.reshape(BATCH, HWA)
    pcls2 = pcls.transpose(0, 2, 1).reshape(BATCH * NC * HWA // 8, 8)
    ptx2 = jnp.pad(
        jnp.pad(ptxywh.reshape(-1), (0, 64)).reshape(5324, 128),
        ((0, 4), (0, 0)))
    gb2 = gboxes.reshape(BATCH, 32)
    glab = glabels.astype(jnp.int32)

    mesh = plsc.VectorSubcoreMesh(core_axis_name="c", subcore_axis_name="s")
    kfn = functools.partial(
        pl.kernel,
        out_type=jax.ShapeDtypeStruct((32, 16), jnp.float32),
        mesh=mesh,
        compiler_params=pltpu.CompilerParams(needs_layout_passes=False,
                                             use_tc_tiling_on_sc=False),
        scratch_types=[
            pltpu.VMEM((10656,), jnp.float32),
            pltpu.VMEM((10656,), jnp.int32),
            pltpu.VMEM((32,), jnp.float32),
            pltpu.VMEM((8,), jnp.int32),
            pltpu.VMEM((16 * NC,), jnp.int32),
            pltpu.VMEM((32,), jnp.int32),
            pltpu.VMEM((16 * NC, 8), jnp.float32),
            pltpu.VMEM((32, 128), jnp.float32),
            pltpu.VMEM((16,), jnp.float32),
            pltpu.SemaphoreType.DMA,
            pltpu.SemaphoreType.DMA,
        ],
    )(_sc_body)
    parts = kfn(pconf2, pcls2, ptx2, gb2, glab)
    p = parts.sum(0)
    return p[0] / BATCH + p[1] / jnp.maximum(p[2], 1.0)
